# baseline (speedup 1.0000x reference)
import functools

import jax
import jax.numpy as jnp
from jax import lax
from jax.experimental import pallas as pl
from jax.experimental.pallas import tpu as pltpu
from jax.experimental.pallas import tpu_sc as plsc

N = 10000
E = 320000
DIM = 64
NB = 12
CUTOFF = 4.0
NTYPES = 10
NGRAPH = 128

NC, NS, LN = 2, 16, 16
NW = NC * NS
EPW = E // NW
KB = 80
NBLK = EPW // KB
NPT = N // NS

BN = 400
GN = N // BN
BE = E // 25

_MESH = plsc.VectorSubcoreMesh(core_axis_name="c", subcore_axis_name="s")
_SC_PARAMS = pltpu.CompilerParams(needs_layout_passes=False,
                                  use_tc_tiling_on_sc=False)


def _sc_dist2(posf, row3d, col3d):
    @functools.partial(
        pl.kernel,
        out_type=[jax.ShapeDtypeStruct((NW, EPW), jnp.float32),
                  jax.ShapeDtypeStruct((2 * N, LN), jnp.float32)],
        mesh=_MESH,
        compiler_params=_SC_PARAMS,
        scratch_types=[
            pltpu.VMEM((3 * N,), jnp.float32),
            pltpu.VMEM((NBLK, KB), jnp.int32),
            pltpu.VMEM((NBLK, KB), jnp.int32),
            pltpu.VMEM((EPW,), jnp.float32),
            pltpu.VMEM((KB, LN), jnp.float32),
            pltpu.VMEM((200, LN), jnp.float32),
            pltpu.VMEM_SHARED((N, LN), jnp.float32),
        ],
    )
    def k(pos_h, row_h, col_h, out_h, deg_h,
          pv, rv, cv, ov, onev, zv, dacc):
        cid = lax.axis_index("c")
        sid = lax.axis_index("s")
        wid = sid * NC + cid

        zero16 = jnp.zeros((LN,), jnp.float32)
        one0 = (lax.iota(jnp.int32, LN) == 0).astype(jnp.float32)

        def zrow(j, carry):
            zv[j, :] = zero16
            return carry

        lax.fori_loop(0, 200, zrow, 0)

        def orow(j, carry):
            onev[j, :] = one0
            return carry

        lax.fori_loop(0, KB, orow, 0)

        @pl.when(sid < 10)
        def _():
            for t5 in range(5):
                pltpu.sync_copy(zv, dacc.at[pl.ds(sid * 1000 + t5 * 200, 200)])

        plsc.subcore_barrier()

        pltpu.sync_copy(pos_h, pv)
        pltpu.sync_copy(row_h.at[wid], rv)
        pltpu.sync_copy(col_h.at[wid], cv)

        def body(j, carry):
            for t in range(KB // LN):
                r3 = rv[j, pl.ds(t * LN, LN)] * 3
                c3 = cv[j, pl.ds(t * LN, LN)] * 3
                dx = plsc.load_gather(pv, [r3]) - plsc.load_gather(pv, [c3])
                dy = (plsc.load_gather(pv, [r3 + 1])
                      - plsc.load_gather(pv, [c3 + 1]))
                dz = (plsc.load_gather(pv, [r3 + 2])
                      - plsc.load_gather(pv, [c3 + 2]))
                ov[pl.ds(j * KB + t * LN, LN)] = dx * dx + dy * dy + dz * dz
            pltpu.sync_copy(onev, dacc.at[cv.at[j]], add=True)
            return carry

        lax.fori_loop(0, NBLK, body, 0)
        pltpu.sync_copy(ov, out_h.at[wid])

        plsc.subcore_barrier()

        @pl.when(sid < 10)
        def _():
            pltpu.sync_copy(dacc.at[pl.ds(sid * 1000, 1000)],
                            deg_h.at[pl.ds(cid * N + sid * 1000, 1000)])

    return k(posf, row3d, col3d)


def _sc_edge(hd_tab, hs_tab, f_edge, row3d, col3d):
    W = DIM

    @functools.partial(
        pl.kernel,
        out_type=jax.ShapeDtypeStruct((2 * N, W), jnp.float32),
        mesh=_MESH,
        compiler_params=_SC_PARAMS,
        scratch_types=[
            pltpu.VMEM((NBLK, KB), jnp.int32),
            pltpu.VMEM((NBLK, KB), jnp.int32),
            [pltpu.VMEM((KB, DIM), jnp.float32)] * 3,
            [pltpu.VMEM((KB, DIM), jnp.float32)] * 3,
            [pltpu.VMEM((KB, DIM), jnp.float32)] * 3,
            [pltpu.VMEM((KB, W), jnp.float32)] * 3,
            pltpu.VMEM_SHARED((N, W), jnp.float32),
            [pltpu.SemaphoreType.DMA] * 9,
            [pltpu.SemaphoreType.DMA] * 3,
        ],
    )
    def k(hd_h, hs_h, f_h, row_h, col_h, out_h,
          rv, cv, hd2, hs2, fv2, tv2, acc, gsem, ssem):
        cid = lax.axis_index("c")
        sid = lax.axis_index("s")
        wid = sid * NC + cid
        ebase = wid * EPW

        zero16 = jnp.zeros((LN,), jnp.float32)

        def zrow(j, carry):
            for t in range(W // LN):
                tv2[2][j, pl.ds(t * LN, LN)] = zero16
            return carry

        lax.fori_loop(0, KB, zrow, 0)

        @pl.when(sid < 10)
        def _():
            for t5 in range(12):
                pltpu.sync_copy(tv2[2],
                                acc.at[pl.ds(sid * 1000 + t5 * KB, KB)])
            pltpu.sync_copy(tv2[2].at[pl.ds(0, 40)],
                            acc.at[pl.ds(sid * 1000 + 960, 40)])

        plsc.subcore_barrier()

        pltpu.sync_copy(row_h.at[wid], rv)
        pltpu.sync_copy(col_h.at[wid], cv)

        def issue(j, s):
            pltpu.async_copy(hd_h.at[cv.at[j]], hd2[s], gsem[3 * s])
            pltpu.async_copy(hs_h.at[rv.at[j]], hs2[s], gsem[3 * s + 1])
            pltpu.async_copy(f_h.at[pl.ds(ebase + j * KB, KB)], fv2[s],
                             gsem[3 * s + 2])

        def wait_in(s):
            pltpu.make_async_copy(hd_h.at[cv.at[0]], hd2[s], gsem[3 * s]).wait()
            pltpu.make_async_copy(hs_h.at[rv.at[0]], hs2[s],
                                  gsem[3 * s + 1]).wait()
            pltpu.make_async_copy(f_h.at[pl.ds(ebase, KB)], fv2[s],
                                  gsem[3 * s + 2]).wait()

        def compute(s):
            hd, hs, fv, tv = hd2[s], hs2[s], fv2[s], tv2[s]

            def erow(e, carry2):
                for t in range(DIM // LN):
                    v = (hd[e, pl.ds(t * LN, LN)]
                         + hs[e, pl.ds(t * LN, LN)]
                         + fv[e, pl.ds(t * LN, LN)])
                    tv[e, pl.ds(t * LN, LN)] = v / (1.0 + jnp.exp(-v))
                return carry2

            lax.fori_loop(0, KB, erow, 0)

        def wait_scat(s):
            pltpu.make_async_copy(tv2[s], acc.at[cv.at[0]], ssem[s]).wait()

        issue(0, 0)
        issue(1, 1)

        def step(j, s):
            wait_in(s)

            @pl.when(j + 2 <= NBLK - 1)
            def _():
                issue(j + 2, (s + 2) % 3)

            @pl.when(j >= 3)
            def _():
                wait_scat(s)

            compute(s)
            pltpu.async_copy(tv2[s], acc.at[cv.at[j]], ssem[s], add=True)

        def tri(i, carry):
            for s in range(3):
                step(3 * i + s, s)
            return carry

        lax.fori_loop(0, NBLK // 3, tri, 0)

        step(jnp.int32(NBLK - 2), 0)
        step(jnp.int32(NBLK - 1), 1)
        for s in range(3):
            wait_scat(s)

        plsc.subcore_barrier()

        @pl.when(sid < 10)
        def _():
            pltpu.sync_copy(acc.at[pl.ds(sid * 1000, 1000)],
                            out_h.at[pl.ds(cid * N + sid * 1000, 1000)])

    return k(hd_tab, hs_tab, f_edge, row3d, col3d)


def _silu(v):
    return v / (1.0 + jnp.exp(-v))


def _dot(a, b):
    return jnp.dot(a, b, preferred_element_type=jnp.float32,
                   precision=lax.Precision.HIGHEST)


def _tc_prep(z3d, emb, w1a, w1b):
    def body(z_ref, emb_ref, wa_ref, wb_ref, x_ref, hd_ref, hs_ref):
        zb = z_ref[0, 0, :]
        oh = (zb[:, None] == lax.broadcasted_iota(jnp.int32, (BN, NTYPES), 1))
        x = _dot(oh.astype(jnp.float32), emb_ref[...])
        x_ref[...] = x
        hd_ref[...] = _dot(x, wa_ref[...])
        hs_ref[...] = _dot(x, wb_ref[...])

    return pl.pallas_call(
        body,
        grid=(GN,),
        in_specs=[
            pl.BlockSpec((1, 1, BN), lambda i: (i, 0, 0)),
            pl.BlockSpec((NTYPES, DIM), lambda i: (0, 0)),
            pl.BlockSpec((DIM, DIM), lambda i: (0, 0)),
            pl.BlockSpec((DIM, DIM), lambda i: (0, 0)),
        ],
        out_specs=[pl.BlockSpec((BN, DIM), lambda i: (i, 0))] * 3,
        out_shape=[jax.ShapeDtypeStruct((N, DIM), jnp.float32)] * 3,
    )(z3d, emb, w1a, w1b)


def _tc_bessel(dist2, freq2, w1c, b1):
    p = 6.0
    ca = -(p + 1.0) * (p + 2.0) / 2.0
    cb = p * (p + 2.0)
    cc = -p * (p + 1.0) / 2.0

    def body(d2_ref, fr_ref, wc_ref, b_ref, f_ref):
        d2 = d2_ref[0, 0, :]
        dist = jnp.sqrt(d2 + 1e-12)
        xx = dist / CUTOFF
        x4 = (xx * xx) * (xx * xx)
        x5 = x4 * xx
        env = jnp.where(xx < 1.0,
                        1.0 / xx + ca * x5 + cb * x5 * xx + cc * x5 * xx * xx,
                        0.0)
        ebt = env[None, :] * jnp.sin(fr_ref[...] * xx[None, :])
        cdims = (((0,), (0,)), ((), ()))
        f_ref[...] = lax.dot_general(ebt, wc_ref[...], cdims,
                                     precision=lax.Precision.HIGHEST,
                                     preferred_element_type=jnp.float32) + b_ref[...]

    return pl.pallas_call(
        body,
        grid=(E // BE,),
        in_specs=[
            pl.BlockSpec((1, 1, BE), lambda i: (i, 0, 0)),
            pl.BlockSpec((NB, 1), lambda i: (0, 0)),
            pl.BlockSpec((NB, DIM), lambda i: (0, 0)),
            pl.BlockSpec((1, DIM), lambda i: (0, 0)),
        ],
        out_specs=pl.BlockSpec((BE, DIM), lambda i: (i, 0)),
        out_shape=jax.ShapeDtypeStruct((E, DIM), jnp.float32),
    )(dist2.reshape(E // BE, 1, BE), freq2, w1c, b1)


def _tc_update0(part, degp, x, w2, b2, ua, ub, ub1, u2, ub2, wa1, wb1):
    def body(p0_ref, p1_ref, d0_ref, d1_ref, x_ref, w2_ref, b2_ref, ua_ref,
             ub_ref, ub1_ref, u2_ref, ub2_ref, wa_ref, wb_ref,
             x1_ref, deg_ref, hd_ref, hs_ref):
        aggh = p0_ref[...] + p1_ref[...]
        deg = d0_ref[:, :1] + d1_ref[:, :1]
        deg_ref[...] = deg
        aggm = _dot(aggh, w2_ref[...]) + deg * b2_ref[...]
        h = _silu(_dot(x_ref[...], ua_ref[...]) + _dot(aggm, ub_ref[...])
                  + ub1_ref[...])
        x1 = x_ref[...] + _dot(h, u2_ref[...]) + ub2_ref[...]
        x1_ref[...] = x1
        hd_ref[...] = _dot(x1, wa_ref[...])
        hs_ref[...] = _dot(x1, wb_ref[...])

    wspec = pl.BlockSpec((DIM, DIM), lambda i: (0, 0))
    bspec = pl.BlockSpec((1, DIM), lambda i: (0, 0))
    nspec = pl.BlockSpec((BN, DIM), lambda i: (i, 0))
    return pl.pallas_call(
        body,
        grid=(GN,),
        in_specs=[
            nspec,
            pl.BlockSpec((BN, DIM), lambda i: (i + GN, 0)),
            pl.BlockSpec((BN, LN), lambda i: (i, 0)),
            pl.BlockSpec((BN, LN), lambda i: (i + GN, 0)),
            nspec, wspec, bspec, wspec, wspec, bspec, wspec, bspec,
            wspec, wspec,
        ],
        out_specs=[nspec, pl.BlockSpec((BN, 1), lambda i: (i, 0)),
                   nspec, nspec],
        out_shape=[
            jax.ShapeDtypeStruct((N, DIM), jnp.float32),
            jax.ShapeDtypeStruct((N, 1), jnp.float32),
            jax.ShapeDtypeStruct((N, DIM), jnp.float32),
            jax.ShapeDtypeStruct((N, DIM), jnp.float32),
        ],
    )(part, part, degp, degp, x, w2, b2, ua, ub, ub1, u2, ub2, wa1, wb1)


def _tc_update1(part, x, deg, w2, b2, ua, ub, ub1, u2, ub2):
    def body(p0_ref, p1_ref, x_ref, deg_ref, w2_ref, b2_ref, ua_ref, ub_ref,
             ub1_ref, u2_ref, ub2_ref, x1_ref):
        aggh = p0_ref[...] + p1_ref[...]
        deg = deg_ref[...]
        aggm = _dot(aggh, w2_ref[...]) + deg * b2_ref[...]
        h = _silu(_dot(x_ref[...], ua_ref[...]) + _dot(aggm, ub_ref[...])
                  + ub1_ref[...])
        x1_ref[...] = x_ref[...] + _dot(h, u2_ref[...]) + ub2_ref[...]

    wspec = pl.BlockSpec((DIM, DIM), lambda i: (0, 0))
    bspec = pl.BlockSpec((1, DIM), lambda i: (0, 0))
    nspec = pl.BlockSpec((BN, DIM), lambda i: (i, 0))
    return pl.pallas_call(
        body,
        grid=(GN,),
        in_specs=[
            pl.BlockSpec((BN, DIM), lambda i: (i, 0)),
            pl.BlockSpec((BN, DIM), lambda i: (i + GN, 0)),
            nspec,
            pl.BlockSpec((BN, 1), lambda i: (i, 0)),
            wspec, bspec, wspec, wspec, bspec, wspec, bspec,
        ],
        out_specs=[nspec],
        out_shape=[jax.ShapeDtypeStruct((N, DIM), jnp.float32)],
    )(part, part, x, deg, w2, b2, ua, ub, ub1, u2, ub2)[0]


def _tc_pool_head(x, batch3d, hw1, hb1, hw2, hb2):
    def body(x_ref, b_ref, w1_ref, b1_ref, w2_ref, b2_ref, out_ref, g_acc):
        i = pl.program_id(0)

        @pl.when(i == 0)
        def _():
            g_acc[...] = jnp.zeros((NGRAPH, DIM), jnp.float32)

        bb = b_ref[0, 0, :]
        oh = (bb[:, None] == lax.broadcasted_iota(jnp.int32, (BN, NGRAPH), 1))
        g_acc[...] += lax.dot_general(oh.astype(jnp.float32), x_ref[...],
                                      (((0,), (0,)), ((), ())),
                                      preferred_element_type=jnp.float32,
                                      precision=lax.Precision.HIGHEST)

        @pl.when(i == GN - 1)
        def _():
            g = g_acc[...]
            out_ref[...] = (_dot(_silu(_dot(g, w1_ref[...]) + b1_ref[...]),
                                 w2_ref[...]) + b2_ref[...])

    return pl.pallas_call(
        body,
        grid=(GN,),
        in_specs=[
            pl.BlockSpec((BN, DIM), lambda i: (i, 0)),
            pl.BlockSpec((1, 1, BN), lambda i: (i, 0, 0)),
            pl.BlockSpec((DIM, DIM), lambda i: (0, 0)),
            pl.BlockSpec((1, DIM), lambda i: (0, 0)),
            pl.BlockSpec((DIM, 1), lambda i: (0, 0)),
            pl.BlockSpec((1, 1), lambda i: (0, 0)),
        ],
        out_specs=pl.BlockSpec((NGRAPH, 1), lambda i: (0, 0)),
        out_shape=jax.ShapeDtypeStruct((NGRAPH, 1), jnp.float32),
        scratch_shapes=[pltpu.VMEM((NGRAPH, DIM), jnp.float32)],
    )(x, batch3d, hw1, hb1, hw2, hb2)


def kernel(z, edge_index, batch, pos, emb, freq,
           msg_W1, msg_b1, msg_W2, msg_b2,
           upd_W1, upd_b1, upd_W2, upd_b2,
           head_W1, head_b1, head_W2, head_b2):
    row = edge_index[0].astype(jnp.int32)
    col = edge_index[1].astype(jnp.int32)
    row3d = row.reshape(NW, NBLK, KB)
    col3d = col.reshape(NW, NBLK, KB)
    posf = pos.reshape(3 * N)
    z3d = z.astype(jnp.int32).reshape(GN, 1, BN)
    batch3d = batch.astype(jnp.int32).reshape(GN, 1, BN)
    freq2 = freq.reshape(NB, 1)

    w1a = [msg_W1[l, :DIM, :] for l in range(2)]
    w1b = [msg_W1[l, DIM:2 * DIM, :] for l in range(2)]
    w1c = [msg_W1[l, 2 * DIM:, :] for l in range(2)]
    b1 = [msg_b1[l].reshape(1, DIM) for l in range(2)]
    b2 = [msg_b2[l].reshape(1, DIM) for l in range(2)]
    ua = [upd_W1[l, :DIM, :] for l in range(2)]
    ub = [upd_W1[l, DIM:, :] for l in range(2)]
    ub1 = [upd_b1[l].reshape(1, DIM) for l in range(2)]
    u2 = [upd_W2[l] for l in range(2)]
    ub2 = [upd_b2[l].reshape(1, DIM) for l in range(2)]

    dist2, degp = _sc_dist2(posf, row3d, col3d)
    x, hd0, hs0 = _tc_prep(z3d, emb, w1a[0], w1b[0])
    f0 = _tc_bessel(dist2, freq2, w1c[0], b1[0])
    f1 = _tc_bessel(dist2, freq2, w1c[1], b1[1])

    part0 = _sc_edge(hd0, hs0, f0, row3d, col3d)
    x1, deg, hd1, hs1 = _tc_update0(part0, degp, x, msg_W2[0], b2[0], ua[0],
                                    ub[0], ub1[0], u2[0], ub2[0],
                                    w1a[1], w1b[1])

    part1 = _sc_edge(hd1, hs1, f1, row3d, col3d)
    x2 = _tc_update1(part1, x1, deg, msg_W2[1], b2[1], ua[1], ub[1],
                     ub1[1], u2[1], ub2[1])

    return _tc_pool_head(x2, batch3d, head_W1, head_b1.reshape(1, DIM),
                         head_W2, head_b2.reshape(1, 1))


# speedup vs baseline: 1.2032x; 1.2032x over previous
import functools

import jax
import jax.numpy as jnp
from jax import lax
from jax.experimental import pallas as pl
from jax.experimental.pallas import tpu as pltpu
from jax.experimental.pallas import tpu_sc as plsc

N = 10000
E = 320000
DIM = 64
NB = 12
CUTOFF = 4.0
NTYPES = 10
NGRAPH = 128

NC, NS, LN = 2, 16, 16
NW = NC * NS
EPW = E // NW
KB = 80
NBLK = EPW // KB
NPT = N // NS

BN = 400
GN = N // BN
BE = E // 25

_MESH = plsc.VectorSubcoreMesh(core_axis_name="c", subcore_axis_name="s")
_SC_PARAMS = pltpu.CompilerParams(needs_layout_passes=False,
                                  use_tc_tiling_on_sc=False)


def _sc_dist2(posf, row3d, col3d):
    @functools.partial(
        pl.kernel,
        out_type=[jax.ShapeDtypeStruct((NW, EPW), jnp.float32),
                  jax.ShapeDtypeStruct((2 * N, LN), jnp.float32)],
        mesh=_MESH,
        compiler_params=_SC_PARAMS,
        scratch_types=[
            pltpu.VMEM((3 * N,), jnp.float32),
            pltpu.VMEM((NBLK, KB), jnp.int32),
            pltpu.VMEM((NBLK, KB), jnp.int32),
            pltpu.VMEM((EPW,), jnp.float32),
            pltpu.VMEM((KB, LN), jnp.float32),
            pltpu.VMEM((200, LN), jnp.float32),
            pltpu.VMEM_SHARED((N, LN), jnp.float32),
        ],
    )
    def k(pos_h, row_h, col_h, out_h, deg_h,
          pv, rv, cv, ov, onev, zv, dacc):
        cid = lax.axis_index("c")
        sid = lax.axis_index("s")
        wid = sid * NC + cid

        zero16 = jnp.zeros((LN,), jnp.float32)
        one0 = (lax.iota(jnp.int32, LN) == 0).astype(jnp.float32)

        def zrow(j, carry):
            zv[j, :] = zero16
            return carry

        lax.fori_loop(0, 200, zrow, 0)

        def orow(j, carry):
            onev[j, :] = one0
            return carry

        lax.fori_loop(0, KB, orow, 0)

        @pl.when(sid < 10)
        def _():
            for t5 in range(5):
                pltpu.sync_copy(zv, dacc.at[pl.ds(sid * 1000 + t5 * 200, 200)])

        plsc.subcore_barrier()

        pltpu.sync_copy(pos_h, pv)
        pltpu.sync_copy(row_h.at[wid], rv)
        pltpu.sync_copy(col_h.at[wid], cv)

        def body(j, carry):
            for t in range(KB // LN):
                r3 = rv[j, pl.ds(t * LN, LN)] * 3
                c3 = cv[j, pl.ds(t * LN, LN)] * 3
                dx = plsc.load_gather(pv, [r3]) - plsc.load_gather(pv, [c3])
                dy = (plsc.load_gather(pv, [r3 + 1])
                      - plsc.load_gather(pv, [c3 + 1]))
                dz = (plsc.load_gather(pv, [r3 + 2])
                      - plsc.load_gather(pv, [c3 + 2]))
                ov[pl.ds(j * KB + t * LN, LN)] = dx * dx + dy * dy + dz * dz
            pltpu.sync_copy(onev, dacc.at[cv.at[j]], add=True)
            return carry

        lax.fori_loop(0, NBLK, body, 0)
        pltpu.sync_copy(ov, out_h.at[wid])

        plsc.subcore_barrier()

        @pl.when(sid < 10)
        def _():
            pltpu.sync_copy(dacc.at[pl.ds(sid * 1000, 1000)],
                            deg_h.at[pl.ds(cid * N + sid * 1000, 1000)])

    return k(posf, row3d, col3d)


def _sc_edge(hd_tab, hs_tab, f_edge, row3d, col3d):
    W = DIM

    @functools.partial(
        pl.kernel,
        out_type=jax.ShapeDtypeStruct((2 * N, W), jnp.float32),
        mesh=_MESH,
        compiler_params=_SC_PARAMS,
        scratch_types=[
            pltpu.VMEM((NBLK, KB), jnp.int32),
            pltpu.VMEM((NBLK, KB), jnp.int32),
            [pltpu.VMEM((KB, DIM), jnp.float32)] * 3,
            [pltpu.VMEM((KB, DIM), jnp.float32)] * 3,
            [pltpu.VMEM((KB, DIM), jnp.float32)] * 3,
            [pltpu.VMEM((KB, W), jnp.float32)] * 3,
            pltpu.VMEM_SHARED((N, W), jnp.float32),
            [pltpu.SemaphoreType.DMA] * 9,
            [pltpu.SemaphoreType.DMA] * 3,
        ],
    )
    def k(hd_h, hs_h, f_h, row_h, col_h, out_h,
          rv, cv, hd2, hs2, fv2, tv2, acc, gsem, ssem):
        cid = lax.axis_index("c")
        sid = lax.axis_index("s")
        wid = sid * NC + cid
        ebase = wid * EPW

        zero16 = jnp.zeros((LN,), jnp.float32)

        def zrow(j, carry):
            for t in range(W // LN):
                tv2[2][j, pl.ds(t * LN, LN)] = zero16
            return carry

        lax.fori_loop(0, KB, zrow, 0)

        @pl.when(sid < 10)
        def _():
            for t5 in range(12):
                pltpu.sync_copy(tv2[2],
                                acc.at[pl.ds(sid * 1000 + t5 * KB, KB)])
            pltpu.sync_copy(tv2[2].at[pl.ds(0, 40)],
                            acc.at[pl.ds(sid * 1000 + 960, 40)])

        plsc.subcore_barrier()

        pltpu.sync_copy(row_h.at[wid], rv)
        pltpu.sync_copy(col_h.at[wid], cv)

        def issue(j, s):
            pltpu.async_copy(hd_h.at[cv.at[j]], hd2[s], gsem[3 * s])
            pltpu.async_copy(hs_h.at[rv.at[j]], hs2[s], gsem[3 * s + 1])
            pltpu.async_copy(f_h.at[pl.ds(ebase + j * KB, KB)], fv2[s],
                             gsem[3 * s + 2])

        def wait_in(s):
            pltpu.make_async_copy(hd_h.at[cv.at[0]], hd2[s], gsem[3 * s]).wait()
            pltpu.make_async_copy(hs_h.at[rv.at[0]], hs2[s],
                                  gsem[3 * s + 1]).wait()
            pltpu.make_async_copy(f_h.at[pl.ds(ebase, KB)], fv2[s],
                                  gsem[3 * s + 2]).wait()

        def compute(s):
            hd, hs, fv, tv = hd2[s], hs2[s], fv2[s], tv2[s]

            def erow(e, carry2):
                for t in range(DIM // LN):
                    v = (hd[e, pl.ds(t * LN, LN)]
                         + hs[e, pl.ds(t * LN, LN)]
                         + fv[e, pl.ds(t * LN, LN)])
                    tv[e, pl.ds(t * LN, LN)] = v / (1.0 + jnp.exp(-v))
                return carry2

            lax.fori_loop(0, KB, erow, 0)

        def wait_scat(s):
            pltpu.make_async_copy(tv2[s], acc.at[cv.at[0]], ssem[s]).wait()

        issue(0, 0)
        issue(1, 1)

        def step(j, s):
            wait_in(s)

            @pl.when(j + 2 <= NBLK - 1)
            def _():
                issue(j + 2, (s + 2) % 3)

            @pl.when(j >= 3)
            def _():
                wait_scat(s)

            compute(s)
            pltpu.async_copy(tv2[s], acc.at[cv.at[j]], ssem[s], add=True)

        def tri(i, carry):
            for s in range(3):
                step(3 * i + s, s)
            return carry

        lax.fori_loop(0, NBLK // 3, tri, 0)

        step(jnp.int32(NBLK - 2), 0)
        step(jnp.int32(NBLK - 1), 1)
        for s in range(3):
            wait_scat(s)

        plsc.subcore_barrier()

        @pl.when(sid < 10)
        def _():
            pltpu.sync_copy(acc.at[pl.ds(sid * 1000, 1000)],
                            out_h.at[pl.ds(cid * N + sid * 1000, 1000)])

    return k(hd_tab, hs_tab, f_edge, row3d, col3d)


def _silu(v):
    return v / (1.0 + jnp.exp(-v))


def _dot(a, b):
    return jnp.dot(a, b, preferred_element_type=jnp.float32,
                   precision=lax.Precision.HIGHEST)


def _tc_prep(z3d, emb, w1a, w1b):
    def body(z_ref, emb_ref, wa_ref, wb_ref, x_ref, hd_ref, hs_ref):
        zb = z_ref[0, 0, :]
        oh = (zb[:, None] == lax.broadcasted_iota(jnp.int32, (BN, NTYPES), 1))
        x = _dot(oh.astype(jnp.float32), emb_ref[...])
        x_ref[...] = x
        hd_ref[...] = _dot(x, wa_ref[...])
        hs_ref[...] = _dot(x, wb_ref[...])

    return pl.pallas_call(
        body,
        grid=(GN,),
        in_specs=[
            pl.BlockSpec((1, 1, BN), lambda i: (i, 0, 0)),
            pl.BlockSpec((NTYPES, DIM), lambda i: (0, 0)),
            pl.BlockSpec((DIM, DIM), lambda i: (0, 0)),
            pl.BlockSpec((DIM, DIM), lambda i: (0, 0)),
        ],
        out_specs=[pl.BlockSpec((BN, DIM), lambda i: (i, 0))] * 3,
        out_shape=[jax.ShapeDtypeStruct((N, DIM), jnp.float32)] * 3,
    )(z3d, emb, w1a, w1b)


def _tc_bessel(dist2, freq2, w1c, b0, b1):
    p = 6.0
    ca = -(p + 1.0) * (p + 2.0) / 2.0
    cb = p * (p + 2.0)
    cc = -p * (p + 1.0) / 2.0

    def body(d2_ref, fr_ref, wc_ref, b0_ref, b1_ref, f0_ref, f1_ref):
        d2 = d2_ref[0, 0, :]
        dist = jnp.sqrt(d2 + 1e-12)
        xx = dist / CUTOFF
        x4 = (xx * xx) * (xx * xx)
        x5 = x4 * xx
        env = jnp.where(xx < 1.0,
                        1.0 / xx + ca * x5 + cb * x5 * xx + cc * x5 * xx * xx,
                        0.0)
        ebt = env[None, :] * jnp.sin(fr_ref[...] * xx[None, :])
        cdims = (((0,), (0,)), ((), ()))
        ff = lax.dot_general(ebt, wc_ref[...], cdims,
                             precision=lax.Precision.HIGHEST,
                             preferred_element_type=jnp.float32)
        f0_ref[...] = ff[:, :DIM] + b0_ref[...]
        f1_ref[...] = ff[:, DIM:] + b1_ref[...]

    return pl.pallas_call(
        body,
        grid=(E // BE,),
        in_specs=[
            pl.BlockSpec((1, 1, BE), lambda i: (i, 0, 0)),
            pl.BlockSpec((NB, 1), lambda i: (0, 0)),
            pl.BlockSpec((NB, 2 * DIM), lambda i: (0, 0)),
            pl.BlockSpec((1, DIM), lambda i: (0, 0)),
            pl.BlockSpec((1, DIM), lambda i: (0, 0)),
        ],
        out_specs=[pl.BlockSpec((BE, DIM), lambda i: (i, 0))] * 2,
        out_shape=[jax.ShapeDtypeStruct((E, DIM), jnp.float32)] * 2,
    )(dist2.reshape(E // BE, 1, BE), freq2, w1c, b0, b1)


def _tc_update0(part, degp, x, w2, b2, ua, ub, ub1, u2, ub2, wa1, wb1):
    def body(p0_ref, p1_ref, d0_ref, d1_ref, x_ref, w2_ref, b2_ref, ua_ref,
             ub_ref, ub1_ref, u2_ref, ub2_ref, wa_ref, wb_ref,
             x1_ref, deg_ref, hd_ref, hs_ref):
        aggh = p0_ref[...] + p1_ref[...]
        deg = d0_ref[:, :1] + d1_ref[:, :1]
        deg_ref[...] = deg
        aggm = _dot(aggh, w2_ref[...]) + deg * b2_ref[...]
        h = _silu(_dot(x_ref[...], ua_ref[...]) + _dot(aggm, ub_ref[...])
                  + ub1_ref[...])
        x1 = x_ref[...] + _dot(h, u2_ref[...]) + ub2_ref[...]
        x1_ref[...] = x1
        hd_ref[...] = _dot(x1, wa_ref[...])
        hs_ref[...] = _dot(x1, wb_ref[...])

    wspec = pl.BlockSpec((DIM, DIM), lambda i: (0, 0))
    bspec = pl.BlockSpec((1, DIM), lambda i: (0, 0))
    nspec = pl.BlockSpec((BN, DIM), lambda i: (i, 0))
    return pl.pallas_call(
        body,
        grid=(GN,),
        in_specs=[
            nspec,
            pl.BlockSpec((BN, DIM), lambda i: (i + GN, 0)),
            pl.BlockSpec((BN, LN), lambda i: (i, 0)),
            pl.BlockSpec((BN, LN), lambda i: (i + GN, 0)),
            nspec, wspec, bspec, wspec, wspec, bspec, wspec, bspec,
            wspec, wspec,
        ],
        out_specs=[nspec, pl.BlockSpec((BN, 1), lambda i: (i, 0)),
                   nspec, nspec],
        out_shape=[
            jax.ShapeDtypeStruct((N, DIM), jnp.float32),
            jax.ShapeDtypeStruct((N, 1), jnp.float32),
            jax.ShapeDtypeStruct((N, DIM), jnp.float32),
            jax.ShapeDtypeStruct((N, DIM), jnp.float32),
        ],
    )(part, part, degp, degp, x, w2, b2, ua, ub, ub1, u2, ub2, wa1, wb1)


def _tc_update1(part, x, deg, w2, b2, ua, ub, ub1, u2, ub2):
    def body(p0_ref, p1_ref, x_ref, deg_ref, w2_ref, b2_ref, ua_ref, ub_ref,
             ub1_ref, u2_ref, ub2_ref, x1_ref):
        aggh = p0_ref[...] + p1_ref[...]
        deg = deg_ref[...]
        aggm = _dot(aggh, w2_ref[...]) + deg * b2_ref[...]
        h = _silu(_dot(x_ref[...], ua_ref[...]) + _dot(aggm, ub_ref[...])
                  + ub1_ref[...])
        x1_ref[...] = x_ref[...] + _dot(h, u2_ref[...]) + ub2_ref[...]

    wspec = pl.BlockSpec((DIM, DIM), lambda i: (0, 0))
    bspec = pl.BlockSpec((1, DIM), lambda i: (0, 0))
    nspec = pl.BlockSpec((BN, DIM), lambda i: (i, 0))
    return pl.pallas_call(
        body,
        grid=(GN,),
        in_specs=[
            pl.BlockSpec((BN, DIM), lambda i: (i, 0)),
            pl.BlockSpec((BN, DIM), lambda i: (i + GN, 0)),
            nspec,
            pl.BlockSpec((BN, 1), lambda i: (i, 0)),
            wspec, bspec, wspec, wspec, bspec, wspec, bspec,
        ],
        out_specs=[nspec],
        out_shape=[jax.ShapeDtypeStruct((N, DIM), jnp.float32)],
    )(part, part, x, deg, w2, b2, ua, ub, ub1, u2, ub2)[0]


def _tc_pool_head(x, batch3d, hw1, hb1, hw2, hb2):
    def body(x_ref, b_ref, w1_ref, b1_ref, w2_ref, b2_ref, out_ref, g_acc):
        i = pl.program_id(0)

        @pl.when(i == 0)
        def _():
            g_acc[...] = jnp.zeros((NGRAPH, DIM), jnp.float32)

        bb = b_ref[0, 0, :]
        oh = (bb[:, None] == lax.broadcasted_iota(jnp.int32, (BN, NGRAPH), 1))
        g_acc[...] += lax.dot_general(oh.astype(jnp.float32), x_ref[...],
                                      (((0,), (0,)), ((), ())),
                                      preferred_element_type=jnp.float32,
                                      precision=lax.Precision.HIGHEST)

        @pl.when(i == GN - 1)
        def _():
            g = g_acc[...]
            out_ref[...] = (_dot(_silu(_dot(g, w1_ref[...]) + b1_ref[...]),
                                 w2_ref[...]) + b2_ref[...])

    return pl.pallas_call(
        body,
        grid=(GN,),
        in_specs=[
            pl.BlockSpec((BN, DIM), lambda i: (i, 0)),
            pl.BlockSpec((1, 1, BN), lambda i: (i, 0, 0)),
            pl.BlockSpec((DIM, DIM), lambda i: (0, 0)),
            pl.BlockSpec((1, DIM), lambda i: (0, 0)),
            pl.BlockSpec((DIM, 1), lambda i: (0, 0)),
            pl.BlockSpec((1, 1), lambda i: (0, 0)),
        ],
        out_specs=pl.BlockSpec((NGRAPH, 1), lambda i: (0, 0)),
        out_shape=jax.ShapeDtypeStruct((NGRAPH, 1), jnp.float32),
        scratch_shapes=[pltpu.VMEM((NGRAPH, DIM), jnp.float32)],
    )(x, batch3d, hw1, hb1, hw2, hb2)


def kernel(z, edge_index, batch, pos, emb, freq,
           msg_W1, msg_b1, msg_W2, msg_b2,
           upd_W1, upd_b1, upd_W2, upd_b2,
           head_W1, head_b1, head_W2, head_b2):
    row = edge_index[0].astype(jnp.int32)
    col = edge_index[1].astype(jnp.int32)
    row3d = row.reshape(NW, NBLK, KB)
    col3d = col.reshape(NW, NBLK, KB)
    posf = pos.reshape(3 * N)
    z3d = z.astype(jnp.int32).reshape(GN, 1, BN)
    batch3d = batch.astype(jnp.int32).reshape(GN, 1, BN)
    freq2 = freq.reshape(NB, 1)

    w1a = [msg_W1[l, :DIM, :] for l in range(2)]
    w1b = [msg_W1[l, DIM:2 * DIM, :] for l in range(2)]
    w1c = [msg_W1[l, 2 * DIM:, :] for l in range(2)]
    b1 = [msg_b1[l].reshape(1, DIM) for l in range(2)]
    b2 = [msg_b2[l].reshape(1, DIM) for l in range(2)]
    ua = [upd_W1[l, :DIM, :] for l in range(2)]
    ub = [upd_W1[l, DIM:, :] for l in range(2)]
    ub1 = [upd_b1[l].reshape(1, DIM) for l in range(2)]
    u2 = [upd_W2[l] for l in range(2)]
    ub2 = [upd_b2[l].reshape(1, DIM) for l in range(2)]

    dist2, degp = _sc_dist2(posf, row3d, col3d)
    x, hd0, hs0 = _tc_prep(z3d, emb, w1a[0], w1b[0])
    f0, f1 = _tc_bessel(dist2, freq2,
                        jnp.concatenate([w1c[0], w1c[1]], axis=1),
                        b1[0], b1[1])

    part0 = _sc_edge(hd0, hs0, f0, row3d, col3d)
    x1, deg, hd1, hs1 = _tc_update0(part0, degp, x, msg_W2[0], b2[0], ua[0],
                                    ub[0], ub1[0], u2[0], ub2[0],
                                    w1a[1], w1b[1])

    part1 = _sc_edge(hd1, hs1, f1, row3d, col3d)
    x2 = _tc_update1(part1, x1, deg, msg_W2[1], b2[1], ua[1], ub[1],
                     ub1[1], u2[1], ub2[1])

    return _tc_pool_head(x2, batch3d, head_W1, head_b1.reshape(1, DIM),
                         head_W2, head_b2.reshape(1, 1))


# speedup vs baseline: 1.5686x; 1.3037x over previous
import functools

import jax
import jax.numpy as jnp
from jax import lax
from jax.experimental import pallas as pl
from jax.experimental.pallas import tpu as pltpu
from jax.experimental.pallas import tpu_sc as plsc

N = 10000
E = 320000
DIM = 64
NB = 12
CUTOFF = 4.0
NTYPES = 10
NGRAPH = 128

NC, NS, LN = 2, 16, 16
NW = NC * NS
EPW = E // NW
KB = 80
NBLK = EPW // KB
NPT = N // NS

BN = 400
GN = N // BN
BE = E // 25

_MESH = plsc.VectorSubcoreMesh(core_axis_name="c", subcore_axis_name="s")
_SC_PARAMS = pltpu.CompilerParams(needs_layout_passes=False,
                                  use_tc_tiling_on_sc=False)


def _sc_dist2(posf, row3d, col3d):
    @functools.partial(
        pl.kernel,
        out_type=[jax.ShapeDtypeStruct((NW, EPW), jnp.float32),
                  jax.ShapeDtypeStruct((2 * N, LN), jnp.float32)],
        mesh=_MESH,
        compiler_params=_SC_PARAMS,
        scratch_types=[
            pltpu.VMEM((3 * N,), jnp.float32),
            pltpu.VMEM((NBLK, KB), jnp.int32),
            pltpu.VMEM((NBLK, KB), jnp.int32),
            pltpu.VMEM((EPW,), jnp.float32),
            pltpu.VMEM((KB, LN), jnp.float32),
            pltpu.VMEM((200, LN), jnp.float32),
            pltpu.VMEM_SHARED((N, LN), jnp.float32),
        ],
    )
    def k(pos_h, row_h, col_h, out_h, deg_h,
          pv, rv, cv, ov, onev, zv, dacc):
        cid = lax.axis_index("c")
        sid = lax.axis_index("s")
        wid = sid * NC + cid

        zero16 = jnp.zeros((LN,), jnp.float32)
        one0 = (lax.iota(jnp.int32, LN) == 0).astype(jnp.float32)

        def zrow(j, carry):
            zv[j, :] = zero16
            return carry

        lax.fori_loop(0, 200, zrow, 0)

        def orow(j, carry):
            onev[j, :] = one0
            return carry

        lax.fori_loop(0, KB, orow, 0)

        @pl.when(sid < 10)
        def _():
            for t5 in range(5):
                pltpu.sync_copy(zv, dacc.at[pl.ds(sid * 1000 + t5 * 200, 200)])

        plsc.subcore_barrier()

        pltpu.sync_copy(pos_h, pv)
        pltpu.sync_copy(row_h.at[wid], rv)
        pltpu.sync_copy(col_h.at[wid], cv)

        def body(j, carry):
            for t in range(KB // LN):
                r3 = rv[j, pl.ds(t * LN, LN)] * 3
                c3 = cv[j, pl.ds(t * LN, LN)] * 3
                dx = plsc.load_gather(pv, [r3]) - plsc.load_gather(pv, [c3])
                dy = (plsc.load_gather(pv, [r3 + 1])
                      - plsc.load_gather(pv, [c3 + 1]))
                dz = (plsc.load_gather(pv, [r3 + 2])
                      - plsc.load_gather(pv, [c3 + 2]))
                ov[pl.ds(j * KB + t * LN, LN)] = dx * dx + dy * dy + dz * dz
            pltpu.sync_copy(onev, dacc.at[cv.at[j]], add=True)
            return carry

        lax.fori_loop(0, NBLK, body, 0)
        pltpu.sync_copy(ov, out_h.at[wid])

        plsc.subcore_barrier()

        @pl.when(sid < 10)
        def _():
            pltpu.sync_copy(dacc.at[pl.ds(sid * 1000, 1000)],
                            deg_h.at[pl.ds(cid * N + sid * 1000, 1000)])

    return k(posf, row3d, col3d)


def _sc_edge(hd_tab, hs_tab, f_edge, row3d, col3d, foff):
    W = DIM

    @functools.partial(
        pl.kernel,
        out_type=jax.ShapeDtypeStruct((2 * N, W), jnp.float32),
        mesh=_MESH,
        compiler_params=_SC_PARAMS,
        scratch_types=[
            pltpu.VMEM((NBLK, KB), jnp.int32),
            pltpu.VMEM((NBLK, KB), jnp.int32),
            [pltpu.VMEM((KB, DIM), jnp.float32)] * 3,
            [pltpu.VMEM((KB, DIM), jnp.float32)] * 3,
            [pltpu.VMEM((KB, DIM), jnp.float32)] * 3,
            [pltpu.VMEM((KB, W), jnp.float32)] * 3,
            pltpu.VMEM_SHARED((N, W), jnp.float32),
            [pltpu.SemaphoreType.DMA] * 9,
            [pltpu.SemaphoreType.DMA] * 3,
        ],
    )
    def k(hd_h, hs_h, f_h, row_h, col_h, out_h,
          rv, cv, hd2, hs2, fv2, tv2, acc, gsem, ssem):
        cid = lax.axis_index("c")
        sid = lax.axis_index("s")
        wid = sid * NC + cid
        ebase = wid * EPW

        zero16 = jnp.zeros((LN,), jnp.float32)

        def zrow(j, carry):
            for t in range(W // LN):
                tv2[2][j, pl.ds(t * LN, LN)] = zero16
            return carry

        lax.fori_loop(0, KB, zrow, 0)

        @pl.when(sid < 10)
        def _():
            for t5 in range(12):
                pltpu.sync_copy(tv2[2],
                                acc.at[pl.ds(sid * 1000 + t5 * KB, KB)])
            pltpu.sync_copy(tv2[2].at[pl.ds(0, 40)],
                            acc.at[pl.ds(sid * 1000 + 960, 40)])

        plsc.subcore_barrier()

        pltpu.sync_copy(row_h.at[wid], rv)
        pltpu.sync_copy(col_h.at[wid], cv)

        def issue(j, s):
            pltpu.async_copy(hd_h.at[cv.at[j]], hd2[s], gsem[3 * s])
            pltpu.async_copy(hs_h.at[rv.at[j]], hs2[s], gsem[3 * s + 1])
            pltpu.async_copy(
                f_h.at[pl.ds(ebase + j * KB, KB), pl.ds(foff, DIM)], fv2[s],
                gsem[3 * s + 2])

        def wait_in(s):
            pltpu.make_async_copy(hd_h.at[cv.at[0]], hd2[s], gsem[3 * s]).wait()
            pltpu.make_async_copy(hs_h.at[rv.at[0]], hs2[s],
                                  gsem[3 * s + 1]).wait()
            pltpu.make_async_copy(
                f_h.at[pl.ds(ebase, KB), pl.ds(foff, DIM)], fv2[s],
                gsem[3 * s + 2]).wait()

        def compute(s):
            hd, hs, fv, tv = hd2[s], hs2[s], fv2[s], tv2[s]

            def erow(e, carry2):
                for t in range(DIM // LN):
                    v = (hd[e, pl.ds(t * LN, LN)]
                         + hs[e, pl.ds(t * LN, LN)]
                         + fv[e, pl.ds(t * LN, LN)])
                    tv[e, pl.ds(t * LN, LN)] = v / (1.0 + jnp.exp(-v))
                return carry2

            lax.fori_loop(0, KB, erow, 0)

        def wait_scat(s):
            pltpu.make_async_copy(tv2[s], acc.at[cv.at[0]], ssem[s]).wait()

        issue(0, 0)
        issue(1, 1)

        def step(j, s):
            wait_in(s)

            @pl.when(j + 2 <= NBLK - 1)
            def _():
                issue(j + 2, (s + 2) % 3)

            @pl.when(j >= 3)
            def _():
                wait_scat(s)

            compute(s)
            pltpu.async_copy(tv2[s], acc.at[cv.at[j]], ssem[s], add=True)

        def tri(i, carry):
            for s in range(3):
                step(3 * i + s, s)
            return carry

        lax.fori_loop(0, NBLK // 3, tri, 0)

        step(jnp.int32(NBLK - 2), 0)
        step(jnp.int32(NBLK - 1), 1)
        for s in range(3):
            wait_scat(s)

        plsc.subcore_barrier()

        @pl.when(sid < 10)
        def _():
            pltpu.sync_copy(acc.at[pl.ds(sid * 1000, 1000)],
                            out_h.at[pl.ds(cid * N + sid * 1000, 1000)])

    return k(hd_tab, hs_tab, f_edge, row3d, col3d)


def _silu(v):
    return v / (1.0 + jnp.exp(-v))


def _dot(a, b):
    return jnp.dot(a, b, preferred_element_type=jnp.float32,
                   precision=lax.Precision.HIGHEST)


def _tc_prep(z3d, emb, w1a, w1b):
    def body(z_ref, emb_ref, wa_ref, wb_ref, x_ref, hd_ref, hs_ref):
        zb = z_ref[0, 0, :]
        oh = (zb[:, None] == lax.broadcasted_iota(jnp.int32, (BN, NTYPES), 1))
        x = _dot(oh.astype(jnp.float32), emb_ref[...])
        x_ref[...] = x
        hd_ref[...] = _dot(x, wa_ref[...])
        hs_ref[...] = _dot(x, wb_ref[...])

    return pl.pallas_call(
        body,
        grid=(GN,),
        in_specs=[
            pl.BlockSpec((1, 1, BN), lambda i: (i, 0, 0)),
            pl.BlockSpec((NTYPES, DIM), lambda i: (0, 0)),
            pl.BlockSpec((DIM, DIM), lambda i: (0, 0)),
            pl.BlockSpec((DIM, DIM), lambda i: (0, 0)),
        ],
        out_specs=[pl.BlockSpec((BN, DIM), lambda i: (i, 0))] * 3,
        out_shape=[jax.ShapeDtypeStruct((N, DIM), jnp.float32)] * 3,
    )(z3d, emb, w1a, w1b)


def _tc_bessel(dist2, freq2, w1c, b01):
    p = 6.0
    ca = -(p + 1.0) * (p + 2.0) / 2.0
    cb = p * (p + 2.0)
    cc = -p * (p + 1.0) / 2.0

    def body(d2_ref, fr_ref, wc_ref, b_ref, f_ref):
        d2 = d2_ref[0, 0, :]
        dist = jnp.sqrt(d2 + 1e-12)
        xx = dist / CUTOFF
        x4 = (xx * xx) * (xx * xx)
        x5 = x4 * xx
        env = jnp.where(xx < 1.0,
                        1.0 / xx + ca * x5 + cb * x5 * xx + cc * x5 * xx * xx,
                        0.0)
        ebt = env[None, :] * jnp.sin(fr_ref[...] * xx[None, :])
        cdims = (((0,), (0,)), ((), ()))
        ff = lax.dot_general(ebt, wc_ref[...], cdims,
                             precision=lax.Precision.HIGHEST,
                             preferred_element_type=jnp.float32)
        f_ref[...] = ff + b_ref[...]

    return pl.pallas_call(
        body,
        grid=(E // BE,),
        in_specs=[
            pl.BlockSpec((1, 1, BE), lambda i: (i, 0, 0)),
            pl.BlockSpec((NB, 1), lambda i: (0, 0)),
            pl.BlockSpec((NB, 2 * DIM), lambda i: (0, 0)),
            pl.BlockSpec((1, 2 * DIM), lambda i: (0, 0)),
        ],
        out_specs=pl.BlockSpec((BE, 2 * DIM), lambda i: (i, 0)),
        out_shape=jax.ShapeDtypeStruct((E, 2 * DIM), jnp.float32),
    )(dist2.reshape(E // BE, 1, BE), freq2, w1c, b01)


def _tc_update0(part, degp, x, w2, b2, ua, ub, ub1, u2, ub2, wa1, wb1):
    def body(p0_ref, p1_ref, d0_ref, d1_ref, x_ref, w2_ref, b2_ref, ua_ref,
             ub_ref, ub1_ref, u2_ref, ub2_ref, wa_ref, wb_ref,
             x1_ref, deg_ref, hd_ref, hs_ref):
        aggh = p0_ref[...] + p1_ref[...]
        deg = d0_ref[:, :1] + d1_ref[:, :1]
        deg_ref[...] = deg
        aggm = _dot(aggh, w2_ref[...]) + deg * b2_ref[...]
        h = _silu(_dot(x_ref[...], ua_ref[...]) + _dot(aggm, ub_ref[...])
                  + ub1_ref[...])
        x1 = x_ref[...] + _dot(h, u2_ref[...]) + ub2_ref[...]
        x1_ref[...] = x1
        hd_ref[...] = _dot(x1, wa_ref[...])
        hs_ref[...] = _dot(x1, wb_ref[...])

    wspec = pl.BlockSpec((DIM, DIM), lambda i: (0, 0))
    bspec = pl.BlockSpec((1, DIM), lambda i: (0, 0))
    nspec = pl.BlockSpec((BN, DIM), lambda i: (i, 0))
    return pl.pallas_call(
        body,
        grid=(GN,),
        in_specs=[
            nspec,
            pl.BlockSpec((BN, DIM), lambda i: (i + GN, 0)),
            pl.BlockSpec((BN, LN), lambda i: (i, 0)),
            pl.BlockSpec((BN, LN), lambda i: (i + GN, 0)),
            nspec, wspec, bspec, wspec, wspec, bspec, wspec, bspec,
            wspec, wspec,
        ],
        out_specs=[nspec, pl.BlockSpec((BN, 1), lambda i: (i, 0)),
                   nspec, nspec],
        out_shape=[
            jax.ShapeDtypeStruct((N, DIM), jnp.float32),
            jax.ShapeDtypeStruct((N, 1), jnp.float32),
            jax.ShapeDtypeStruct((N, DIM), jnp.float32),
            jax.ShapeDtypeStruct((N, DIM), jnp.float32),
        ],
    )(part, part, degp, degp, x, w2, b2, ua, ub, ub1, u2, ub2, wa1, wb1)


def _tc_update1(part, x, deg, w2, b2, ua, ub, ub1, u2, ub2):
    def body(p0_ref, p1_ref, x_ref, deg_ref, w2_ref, b2_ref, ua_ref, ub_ref,
             ub1_ref, u2_ref, ub2_ref, x1_ref):
        aggh = p0_ref[...] + p1_ref[...]
        deg = deg_ref[...]
        aggm = _dot(aggh, w2_ref[...]) + deg * b2_ref[...]
        h = _silu(_dot(x_ref[...], ua_ref[...]) + _dot(aggm, ub_ref[...])
                  + ub1_ref[...])
        x1_ref[...] = x_ref[...] + _dot(h, u2_ref[...]) + ub2_ref[...]

    wspec = pl.BlockSpec((DIM, DIM), lambda i: (0, 0))
    bspec = pl.BlockSpec((1, DIM), lambda i: (0, 0))
    nspec = pl.BlockSpec((BN, DIM), lambda i: (i, 0))
    return pl.pallas_call(
        body,
        grid=(GN,),
        in_specs=[
            pl.BlockSpec((BN, DIM), lambda i: (i, 0)),
            pl.BlockSpec((BN, DIM), lambda i: (i + GN, 0)),
            nspec,
            pl.BlockSpec((BN, 1), lambda i: (i, 0)),
            wspec, bspec, wspec, wspec, bspec, wspec, bspec,
        ],
        out_specs=[nspec],
        out_shape=[jax.ShapeDtypeStruct((N, DIM), jnp.float32)],
    )(part, part, x, deg, w2, b2, ua, ub, ub1, u2, ub2)[0]


def _tc_pool_head(x, batch3d, hw1, hb1, hw2, hb2):
    def body(x_ref, b_ref, w1_ref, b1_ref, w2_ref, b2_ref, out_ref, g_acc):
        i = pl.program_id(0)

        @pl.when(i == 0)
        def _():
            g_acc[...] = jnp.zeros((NGRAPH, DIM), jnp.float32)

        bb = b_ref[0, 0, :]
        oh = (bb[:, None] == lax.broadcasted_iota(jnp.int32, (BN, NGRAPH), 1))
        g_acc[...] += lax.dot_general(oh.astype(jnp.float32), x_ref[...],
                                      (((0,), (0,)), ((), ())),
                                      preferred_element_type=jnp.float32,
                                      precision=lax.Precision.HIGHEST)

        @pl.when(i == GN - 1)
        def _():
            g = g_acc[...]
            out_ref[...] = (_dot(_silu(_dot(g, w1_ref[...]) + b1_ref[...]),
                                 w2_ref[...]) + b2_ref[...])

    return pl.pallas_call(
        body,
        grid=(GN,),
        in_specs=[
            pl.BlockSpec((BN, DIM), lambda i: (i, 0)),
            pl.BlockSpec((1, 1, BN), lambda i: (i, 0, 0)),
            pl.BlockSpec((DIM, DIM), lambda i: (0, 0)),
            pl.BlockSpec((1, DIM), lambda i: (0, 0)),
            pl.BlockSpec((DIM, 1), lambda i: (0, 0)),
            pl.BlockSpec((1, 1), lambda i: (0, 0)),
        ],
        out_specs=pl.BlockSpec((NGRAPH, 1), lambda i: (0, 0)),
        out_shape=jax.ShapeDtypeStruct((NGRAPH, 1), jnp.float32),
        scratch_shapes=[pltpu.VMEM((NGRAPH, DIM), jnp.float32)],
    )(x, batch3d, hw1, hb1, hw2, hb2)


def kernel(z, edge_index, batch, pos, emb, freq,
           msg_W1, msg_b1, msg_W2, msg_b2,
           upd_W1, upd_b1, upd_W2, upd_b2,
           head_W1, head_b1, head_W2, head_b2):
    row = edge_index[0].astype(jnp.int32)
    col = edge_index[1].astype(jnp.int32)
    row3d = row.reshape(NW, NBLK, KB)
    col3d = col.reshape(NW, NBLK, KB)
    posf = pos.reshape(3 * N)
    z3d = z.astype(jnp.int32).reshape(GN, 1, BN)
    batch3d = batch.astype(jnp.int32).reshape(GN, 1, BN)
    freq2 = freq.reshape(NB, 1)

    w1a = [msg_W1[l, :DIM, :] for l in range(2)]
    w1b = [msg_W1[l, DIM:2 * DIM, :] for l in range(2)]
    w1c = [msg_W1[l, 2 * DIM:, :] for l in range(2)]
    b1 = [msg_b1[l].reshape(1, DIM) for l in range(2)]
    b2 = [msg_b2[l].reshape(1, DIM) for l in range(2)]
    ua = [upd_W1[l, :DIM, :] for l in range(2)]
    ub = [upd_W1[l, DIM:, :] for l in range(2)]
    ub1 = [upd_b1[l].reshape(1, DIM) for l in range(2)]
    u2 = [upd_W2[l] for l in range(2)]
    ub2 = [upd_b2[l].reshape(1, DIM) for l in range(2)]

    dist2, degp = _sc_dist2(posf, row3d, col3d)
    x, hd0, hs0 = _tc_prep(z3d, emb, w1a[0], w1b[0])
    f01 = _tc_bessel(dist2, freq2,
                     jnp.concatenate([w1c[0], w1c[1]], axis=1),
                     jnp.concatenate([b1[0], b1[1]], axis=1))

    part0 = _sc_edge(hd0, hs0, f01, row3d, col3d, 0)
    x1, deg, hd1, hs1 = _tc_update0(part0, degp, x, msg_W2[0], b2[0], ua[0],
                                    ub[0], ub1[0], u2[0], ub2[0],
                                    w1a[1], w1b[1])

    part1 = _sc_edge(hd1, hs1, f01, row3d, col3d, DIM)
    x2 = _tc_update1(part1, x1, deg, msg_W2[1], b2[1], ua[1], ub[1],
                     ub1[1], u2[1], ub2[1])

    return _tc_pool_head(x2, batch3d, head_W1, head_b1.reshape(1, DIM),
                         head_W2, head_b2.reshape(1, 1))
